# baseline (speedup 1.0000x reference)
import functools

import jax
import jax.numpy as jnp
from jax import lax
from jax.experimental import pallas as pl
from jax.experimental.pallas import tpu as pltpu
from jax.experimental.pallas import tpu_sc as plsc

_B = 8
_N = 8192
_C = 512
_K = 1024
_LANES = 16
_CHUNKS = _N // _LANES


def _tc_body(x_ref, s_ref, i_ref):
    x = x_ref[0]
    m = jnp.max(x, axis=-1, keepdims=True)
    iota = lax.broadcasted_iota(jnp.int32, x.shape, 1).astype(jnp.float32)
    cand = jnp.where(x == m, iota, jnp.float32(_C))
    idx = jnp.min(cand, axis=-1, keepdims=True)
    s_ref[...] = m
    i_ref[...] = idx.astype(jnp.int32)


def _tc_scores(similarity, blk=4096):
    nb = (_B * _N) // blk
    sim = similarity.reshape(nb, blk, _C)
    scores, idx = pl.pallas_call(
        _tc_body,
        grid=(nb,),
        in_specs=[pl.BlockSpec((1, blk, _C), lambda i: (i, 0, 0))],
        out_specs=[
            pl.BlockSpec((blk, 1), lambda i: (i, 0)),
            pl.BlockSpec((blk, 1), lambda i: (i, 0)),
        ],
        out_shape=[
            jax.ShapeDtypeStruct((nb * blk, 1), jnp.float32),
            jax.ShapeDtypeStruct((nb * blk, 1), jnp.int32),
        ],
    )(sim)
    return scores.reshape(_B, _N), idx.reshape(_B, _N)


_INT_MIN = -(2 ** 31)


def _sc_mask_body(scores_hbm, out_hbm, row_v, key_v, hist_v, bint_v, mask_v):
    wid = lax.axis_index("s") * 2 + lax.axis_index("c")

    @pl.when(wid < _B)
    def _():
        lane = lax.broadcasted_iota(jnp.int32, (_LANES,), 0)
        ones = jnp.ones((_LANES,), jnp.int32)
        zeros16 = jnp.zeros((_LANES,), jnp.int32)
        int_min = jnp.int32(_INT_MIN)

        pltpu.sync_copy(scores_hbm.at[wid], row_v)

        def zero_hist(i, c):
            hist_v[pl.ds(i * _LANES, _LANES)] = zeros16
            return c

        lax.fori_loop(0, 256, zero_hist, jnp.int32(0))

        def build(i, c):
            b = lax.bitcast_convert_type(
                row_v[pl.ds(i * _LANES, _LANES)], jnp.int32)
            skey = b ^ lax.shift_right_logical(
                lax.shift_right_arithmetic(b, 31), 1)
            key_v[pl.ds(i * _LANES, _LANES)] = skey
            byte = (lax.shift_right_logical(skey, 24) & 0xFF) ^ 0x80
            plsc.addupdate_scatter(hist_v, [byte * _LANES + lane], ones)
            return c

        lax.fori_loop(0, _CHUNKS, build, jnp.int32(0))

        def bin_scan(need):
            def lanesum(g, c):
                acc = zeros16
                base = g * 256
                for l in range(_LANES):
                    acc = acc + plsc.load_gather(
                        hist_v, [base + lane * _LANES + l])
                bint_v[pl.ds(g * _LANES, _LANES)] = acc
                return c

            lax.fori_loop(0, 16, lanesum, jnp.int32(0))

            def cross(j, carry):
                run, t, s = carry
                g = 15 - j
                c_desc = lax.rev(bint_v[pl.ds(g * _LANES, _LANES)], (0,))
                csum = plsc.cumsum(c_desc)
                prev = csum - c_desc
                sb = run + prev
                hit = (sb < need) & ((sb + c_desc) >= need)
                bins_desc = g * _LANES + 15 - lane
                t = jnp.maximum(t, jnp.max(jnp.where(hit, bins_desc, -1)))
                s = jnp.maximum(s, jnp.max(jnp.where(hit, sb, -1)))
                return (run + csum[15], t, s)

            _, t, s = lax.fori_loop(
                0, 16, cross, (jnp.int32(0), jnp.int32(-1), jnp.int32(-1)))
            return t, s

        t0, s0 = bin_scan(jnp.int32(_K))
        need = _K - s0
        prefix_u = t0 << 24

        for p in (1, 2, 3):
            sh = 24 - 8 * p
            psh = 32 - 8 * p
            lax.fori_loop(0, 256, zero_hist, jnp.int32(0))
            pref_cmp = lax.shift_right_logical(prefix_u, psh)

            def histp(i, c, sh=sh, psh=psh, pref_cmp=pref_cmp):
                ukey = key_v[pl.ds(i * _LANES, _LANES)] ^ int_min
                cand = lax.shift_right_logical(ukey, psh) == pref_cmp
                byte = lax.shift_right_logical(ukey, sh) & 0xFF
                plsc.addupdate_scatter(
                    hist_v, [byte * _LANES + lane], ones, mask=cand)
                return c

            lax.fori_loop(0, _CHUNKS, histp, jnp.int32(0))
            t, s = bin_scan(need)
            need = need - s
            prefix_u = prefix_u | (t << sh)

        thr = prefix_u ^ int_min

        def mask_body(i, carry):
            k = key_v[pl.ds(i * _LANES, _LANES)]
            gt = k > thr
            eq = k == thr
            csum = plsc.cumsum(eq.astype(jnp.int32))
            sel = gt | (eq & ((carry + csum) <= need))
            mask_v[pl.ds(i * _LANES, _LANES)] = jnp.where(
                sel, jnp.float32(0.0), jnp.float32(1.0))
            return carry + csum[15]

        lax.fori_loop(0, _CHUNKS, mask_body, jnp.int32(0))
        pltpu.sync_copy(mask_v, out_hbm.at[wid])


@functools.lru_cache(maxsize=1)
def _sc_topk_mask():
    return pl.kernel(
        _sc_mask_body,
        out_type=jax.ShapeDtypeStruct((_B, _N), jnp.float32),
        mesh=plsc.VectorSubcoreMesh(core_axis_name="c", subcore_axis_name="s"),
        scratch_types=[
            pltpu.VMEM((_N,), jnp.float32),
            pltpu.VMEM((_N,), jnp.int32),
            pltpu.VMEM((256 * _LANES,), jnp.int32),
            pltpu.VMEM((256,), jnp.int32),
            pltpu.VMEM((_N,), jnp.float32),
        ],
        compiler_params=pltpu.CompilerParams(needs_layout_passes=False),
    )


def kernel(importance, similarity, compressed_map):
    scores, ms_idx = _tc_scores(similarity)
    mask = _sc_topk_mask()(scores)
    return (mask[..., None], ms_idx)


# speedup vs baseline: 1.0501x; 1.0501x over previous
import functools

import jax
import jax.numpy as jnp
from jax import lax
from jax.experimental import pallas as pl
from jax.experimental.pallas import tpu as pltpu
from jax.experimental.pallas import tpu_sc as plsc

_B = 8
_N = 8192
_C = 512
_K = 1024
_LANES = 16
_CHUNKS = _N // _LANES


def _tc_body(x_ref, s_ref, i_ref):
    x = x_ref[0]
    m = jnp.max(x, axis=-1, keepdims=True)
    iota = lax.broadcasted_iota(jnp.int32, x.shape, 1).astype(jnp.float32)
    cand = jnp.where(x == m, iota, jnp.float32(_C))
    idx = jnp.min(cand, axis=-1, keepdims=True)
    s_ref[...] = m
    i_ref[...] = idx.astype(jnp.int32)


def _tc_scores(similarity, blk=4096):
    nb = (_B * _N) // blk
    sim = similarity.reshape(nb, blk, _C)
    scores, idx = pl.pallas_call(
        _tc_body,
        grid=(nb,),
        in_specs=[pl.BlockSpec((1, blk, _C), lambda i: (i, 0, 0))],
        out_specs=[
            pl.BlockSpec((blk, 1), lambda i: (i, 0)),
            pl.BlockSpec((blk, 1), lambda i: (i, 0)),
        ],
        out_shape=[
            jax.ShapeDtypeStruct((nb * blk, 1), jnp.float32),
            jax.ShapeDtypeStruct((nb * blk, 1), jnp.int32),
        ],
    )(sim)
    return scores.reshape(_B, _N), idx.reshape(_B, _N)


_INT_MIN = -(2 ** 31)


def _sc_mask_body(scores_hbm, out_hbm, row_v, key_v, hist_v, bint_v, mask_v):
    wid = lax.axis_index("s") * 2 + lax.axis_index("c")

    @pl.when(wid < _B)
    def _():
        lane = lax.broadcasted_iota(jnp.int32, (_LANES,), 0)
        ones = jnp.ones((_LANES,), jnp.int32)
        zeros16 = jnp.zeros((_LANES,), jnp.int32)
        int_min = jnp.int32(_INT_MIN)

        pltpu.sync_copy(scores_hbm.at[wid], row_v)

        def zero_hist(i, c):
            for u in range(4):
                hist_v[pl.ds((i * 4 + u) * _LANES, _LANES)] = zeros16
            return c

        lax.fori_loop(0, 64, zero_hist, jnp.int32(0))

        def build(i, c):
            for u in range(4):
                j = i * 4 + u
                b = lax.bitcast_convert_type(
                    row_v[pl.ds(j * _LANES, _LANES)], jnp.int32)
                skey = b ^ lax.shift_right_logical(
                    lax.shift_right_arithmetic(b, 31), 1)
                key_v[pl.ds(j * _LANES, _LANES)] = skey
                byte = (lax.shift_right_logical(skey, 24) & 0xFF) ^ 0x80
                plsc.addupdate_scatter(hist_v, [byte * _LANES + lane], ones)
            return c

        lax.fori_loop(0, _CHUNKS // 4, build, jnp.int32(0))

        def bin_scan(need):
            def lanesum(g, c):
                acc = zeros16
                base = g * 256
                for l in range(_LANES):
                    acc = acc + plsc.load_gather(
                        hist_v, [base + lane * _LANES + l])
                bint_v[pl.ds(g * _LANES, _LANES)] = acc
                return c

            lax.fori_loop(0, 16, lanesum, jnp.int32(0))

            def cross(j, carry):
                run, t, s = carry
                g = 15 - j
                c_desc = lax.rev(bint_v[pl.ds(g * _LANES, _LANES)], (0,))
                csum = plsc.cumsum(c_desc)
                prev = csum - c_desc
                sb = run + prev
                hit = (sb < need) & ((sb + c_desc) >= need)
                bins_desc = g * _LANES + 15 - lane
                t = jnp.maximum(t, jnp.max(jnp.where(hit, bins_desc, -1)))
                s = jnp.maximum(s, jnp.max(jnp.where(hit, sb, -1)))
                return (run + csum[15], t, s)

            _, t, s = lax.fori_loop(
                0, 16, cross, (jnp.int32(0), jnp.int32(-1), jnp.int32(-1)))
            return t, s

        t0, s0 = bin_scan(jnp.int32(_K))
        need = _K - s0
        prefix_u = t0 << 24

        for p in (1, 2, 3):
            sh = 24 - 8 * p
            psh = 32 - 8 * p
            lax.fori_loop(0, 64, zero_hist, jnp.int32(0))
            pref_cmp = lax.shift_right_logical(prefix_u, psh)

            def histp(i, c, sh=sh, psh=psh, pref_cmp=pref_cmp):
                for u in range(4):
                    j = i * 4 + u
                    ukey = key_v[pl.ds(j * _LANES, _LANES)] ^ int_min
                    cand = lax.shift_right_logical(ukey, psh) == pref_cmp
                    byte = lax.shift_right_logical(ukey, sh) & 0xFF
                    plsc.addupdate_scatter(
                        hist_v, [byte * _LANES + lane], ones, mask=cand)
                return c

            lax.fori_loop(0, _CHUNKS // 4, histp, jnp.int32(0))
            t, s = bin_scan(need)
            need = need - s
            prefix_u = prefix_u | (t << sh)

        thr = prefix_u ^ int_min

        def mask_body(i, carry):
            for u in range(4):
                j = i * 4 + u
                k = key_v[pl.ds(j * _LANES, _LANES)]
                gt = k > thr
                eq = k == thr
                csum = plsc.cumsum(eq.astype(jnp.int32))
                sel = gt | (eq & ((carry + csum) <= need))
                mask_v[pl.ds(j * _LANES, _LANES)] = jnp.where(
                    sel, jnp.float32(0.0), jnp.float32(1.0))
                carry = carry + csum[15]
            return carry

        lax.fori_loop(0, _CHUNKS // 4, mask_body, jnp.int32(0))
        pltpu.sync_copy(mask_v, out_hbm.at[wid])


@functools.lru_cache(maxsize=1)
def _sc_topk_mask():
    return pl.kernel(
        _sc_mask_body,
        out_type=jax.ShapeDtypeStruct((_B, _N), jnp.float32),
        mesh=plsc.VectorSubcoreMesh(core_axis_name="c", subcore_axis_name="s"),
        scratch_types=[
            pltpu.VMEM((_N,), jnp.float32),
            pltpu.VMEM((_N,), jnp.int32),
            pltpu.VMEM((256 * _LANES,), jnp.int32),
            pltpu.VMEM((256,), jnp.int32),
            pltpu.VMEM((_N,), jnp.float32),
        ],
        compiler_params=pltpu.CompilerParams(needs_layout_passes=False),
    )


def kernel(importance, similarity, compressed_map):
    scores, ms_idx = _tc_scores(similarity)
    mask = _sc_topk_mask()(scores)
    return (mask[..., None], ms_idx)
